# baseline (speedup 1.0000x reference)
import functools

import jax
import jax.numpy as jnp
from jax import lax
from jax.experimental import pallas as pl
from jax.experimental.pallas import tpu as pltpu
from jax.experimental.pallas import tpu_sc as plsc

NC = 2
NS = 16
L = 16
NW = NC * NS
ZB = 80
EK = 128

_GDN = lax.GatherDimensionNumbers(
    offset_dims=(), collapsed_slice_dims=(0,), start_index_map=(0,))


def _perm(v, idx):
  return lax.gather(v, idx[:, None], _GDN, (1,),
                    mode=lax.GatherScatterMode.PROMISE_IN_BOUNDS)


def _lane_sum(v):
  lanes = lax.iota(jnp.int32, L)
  for k in (1, 2, 4, 8):
    v = v + _perm(v, lanes ^ k)
  return v


def _bcast_lane(v, e):
  return _perm(v, jnp.full((L,), e, jnp.int32))


def _rsqrt_vec(v):
  i = lax.bitcast_convert_type(v, jnp.int32)
  i = jnp.int32(0x5F3759DF) - (i >> 1)
  y = lax.bitcast_convert_type(i, jnp.float32)
  for _ in range(3):
    y = y * (1.5 - 0.5 * v * y * y)
  return y


def _mesh():
  return plsc.VectorSubcoreMesh(core_axis_name="c", subcore_axis_name="s",
                                num_cores=NC, num_subcores=NS)


_CPARAMS = pltpu.CompilerParams(needs_layout_passes=False)


def _make_k1(n, e, d):
  iters = e // EK // NW
  zit = n // ZB // NS
  jv = d // L

  @functools.partial(
      pl.kernel,
      out_type=(
          jax.ShapeDtypeStruct((NC * n, d), jnp.float32),
          jax.ShapeDtypeStruct((NW * n,), jnp.float32),
      ),
      mesh=_mesh(),
      compiler_params=_CPARAMS,
      scratch_types=[
          pltpu.VMEM((EK,), jnp.int32),
          pltpu.VMEM((EK,), jnp.int32),
          pltpu.VMEM((EK,), jnp.int32),
          pltpu.VMEM((EK,), jnp.int32),
          pltpu.VMEM((EK,), jnp.float32),
          pltpu.VMEM((EK,), jnp.float32),
          pltpu.VMEM((EK, d), jnp.float32),
          pltpu.VMEM((EK, d), jnp.float32),
          pltpu.VMEM((ZB,), jnp.int32),
          pltpu.VMEM((n,), jnp.float32),
          pltpu.VMEM_SHARED((n, d), jnp.float32),
          pltpu.SemaphoreType.DMA,
          pltpu.SemaphoreType.DMA,
      ],
  )
  def k1(x_hbm, src_hbm, dst_hbm, w_hbm, px_hbm, pw_hbm,
         srcv0, srcv1, dstv0, dstv1, wv0, wv1, rows0, rows1,
         zidx, awacc, accs, sem0, sem1):
    cid = lax.axis_index("c")
    sid = lax.axis_index("s")
    wid = sid * NC + cid
    lanes = lax.iota(jnp.int32, L)
    srcvs = (srcv0, srcv1)
    dstvs = (dstv0, dstv1)
    wvs = (wv0, wv1)
    rowss = (rows0, rows1)
    sems = (sem0, sem1)

    def zdbuf(r, _):
      for j in range(jv):
        rows0[r, pl.ds(j * L, L)] = jnp.zeros((L,), jnp.float32)
      return 0
    lax.fori_loop(0, ZB, zdbuf, 0)

    def zaw(r, _):
      awacc[pl.ds(r * L, L)] = jnp.zeros((L,), jnp.float32)
      return 0
    lax.fori_loop(0, n // L, zaw, 0)

    def set_zidx(base):
      for g in range(ZB // L):
        zidx[pl.ds(g * L, L)] = base + g * L + lanes

    def zchunk(z, _):
      base = (z * NS + sid) * ZB
      set_zidx(base)
      pltpu.sync_copy(rows0.at[pl.ds(0, ZB)], accs.at[zidx])
      return 0
    lax.fori_loop(0, zit, zchunk, 0)
    plsc.subcore_barrier()

    def idx_load(i, q):
      off = (i * NW + wid) * EK
      pltpu.sync_copy(src_hbm.at[pl.ds(off, EK)], srcvs[q])
      pltpu.sync_copy(dst_hbm.at[pl.ds(off, EK)], dstvs[q])
      pltpu.sync_copy(w_hbm.at[pl.ds(off, EK)], wvs[q])

    def issue(q):
      pltpu.async_copy(x_hbm.at[srcvs[q]], rowss[q], sems[q])

    def wait_gather(q):
      pltpu.make_async_copy(x_hbm.at[srcvs[q]], rowss[q], sems[q]).wait()

    def consume(q):
      rows = rowss[q]
      for g in range(EK // L):
        awv = 1.0 - wvs[q][pl.ds(g * L, L)]
        dst16 = dstvs[q][pl.ds(g * L, L)]

        def edge_body(e16, _):
          erow = g * L + e16
          awb = _bcast_lane(awv, e16)
          for j in range(jv):
            sl = pl.ds(j * L, L)
            rows[erow, sl] = rows[erow, sl] * awb
          plsc.addupdate_scatter(awacc, [dst16], awv, mask=lanes == e16)
          return 0
        lax.fori_loop(0, L, edge_body, 0)

      pltpu.sync_copy(rows, accs.at[dstvs[q]], add=True)

    idx_load(0, 0)
    issue(0)
    idx_load(1, 1)
    issue(1)

    def pipe_body(io, _):
      for q in (0, 1):
        i = io * 2 + q
        wait_gather(q)
        consume(q)
        idx_load(i + 2, q)
        issue(q)
      return 0
    lax.fori_loop(0, (iters - 2) // 2, pipe_body, 0)

    for q in (0, 1):
      wait_gather(q)
      consume(q)

    plsc.subcore_barrier()

    def dchunk(z, _):
      base = (z * NS + sid) * ZB
      set_zidx(base)
      pltpu.async_copy(accs.at[zidx], rows0.at[pl.ds(0, ZB)], sem0).wait()
      pltpu.sync_copy(rows0.at[pl.ds(0, ZB)],
                      px_hbm.at[pl.ds(cid * n + base, ZB)])
      return 0
    lax.fori_loop(0, zit, dchunk, 0)

    pltpu.sync_copy(awacc, pw_hbm.at[pl.ds(wid * n, n)])

  return k1


def _make_k2(n, d):
  iters = n // ZB // NW
  jv = d // L

  @functools.partial(
      pl.kernel,
      out_type=(
          jax.ShapeDtypeStruct((n, d), jnp.float32),
          jax.ShapeDtypeStruct((n,), jnp.float32),
      ),
      mesh=_mesh(),
      compiler_params=_CPARAMS,
      scratch_types=[
          pltpu.VMEM((ZB, d), jnp.float32),
          pltpu.VMEM((ZB, d), jnp.float32),
          pltpu.VMEM((ZB, d), jnp.float32),
          pltpu.VMEM((NW * ZB,), jnp.float32),
          pltpu.VMEM((ZB, d), jnp.float32),
          pltpu.VMEM((ZB,), jnp.float32),
      ],
  )
  def k2(x_hbm, px_hbm, pw_hbm, nx_hbm, rinv_hbm,
         xv, p0v, p1v, pwv, outv, rv):
    cid = lax.axis_index("c")
    sid = lax.axis_index("s")
    wid = sid * NC + cid
    lanes = lax.iota(jnp.int32, L)

    def chunk_body(i, _):
      ro = (i * NW + wid) * ZB
      sl_rows = pl.ds(ro, ZB)
      pltpu.sync_copy(x_hbm.at[sl_rows], xv)
      pltpu.sync_copy(px_hbm.at[pl.ds(ro, ZB)], p0v)
      pltpu.sync_copy(px_hbm.at[pl.ds(n + ro, ZB)], p1v)
      for t in range(NW):
        pltpu.sync_copy(pw_hbm.at[pl.ds(t * n + ro, ZB)],
                        pwv.at[pl.ds(t * ZB, ZB)])

      for g in range(ZB // L):
        saw = jnp.zeros((L,), jnp.float32)
        for t in range(NW):
          saw = saw + pwv[pl.ds(t * ZB + g * L, L)]

        def node_body(e16, rpack):
          r = g * L + e16
          den = 1.0 + _bcast_lane(saw, e16)
          ss = jnp.zeros((L,), jnp.float32)
          for j in range(jv):
            sl = pl.ds(j * L, L)
            num = (xv[r, sl] + p0v[r, sl] + p1v[r, sl]) / den
            outv[r, sl] = num
            ss = ss + num * num
          rr = jnp.minimum(_rsqrt_vec(_lane_sum(ss)), 1e8)
          return jnp.where(lanes == e16, rr, rpack)
        rpack = lax.fori_loop(0, L, node_body, jnp.zeros((L,), jnp.float32))
        rv[pl.ds(g * L, L)] = rpack

      pltpu.sync_copy(outv, nx_hbm.at[sl_rows])
      pltpu.sync_copy(rv, rinv_hbm.at[sl_rows])
      return 0
    lax.fori_loop(0, iters, chunk_body, 0)

  return k2


def _make_k3(n, e, d):
  iters = e // EK // NW
  jv = d // L

  @functools.partial(
      pl.kernel,
      out_type=jax.ShapeDtypeStruct((e,), jnp.float32),
      mesh=_mesh(),
      compiler_params=_CPARAMS,
      scratch_types=[
          pltpu.VMEM((EK,), jnp.int32),
          pltpu.VMEM((EK,), jnp.int32),
          pltpu.VMEM((EK,), jnp.int32),
          pltpu.VMEM((EK,), jnp.int32),
          pltpu.VMEM((EK,), jnp.float32),
          pltpu.VMEM((EK,), jnp.float32),
          pltpu.VMEM((EK, d), jnp.float32),
          pltpu.VMEM((EK, d), jnp.float32),
          pltpu.VMEM((EK, d), jnp.float32),
          pltpu.VMEM((EK, d), jnp.float32),
          pltpu.VMEM((EK,), jnp.float32),
          pltpu.VMEM((n,), jnp.float32),
          pltpu.SemaphoreType.DMA,
          pltpu.SemaphoreType.DMA,
      ],
  )
  def k3(nx_hbm, rinv_hbm, src_hbm, dst_hbm, w_hbm, neww_hbm,
         srcv0, srcv1, dstv0, dstv1, wv0, wv1, xs0, xs1, xd0, xd1,
         outv, rfull, sem0, sem1):
    cid = lax.axis_index("c")
    sid = lax.axis_index("s")
    wid = sid * NC + cid
    lanes = lax.iota(jnp.int32, L)
    srcvs = (srcv0, srcv1)
    dstvs = (dstv0, dstv1)
    wvs = (wv0, wv1)
    xss = (xs0, xs1)
    xds = (xd0, xd1)
    sems = (sem0, sem1)

    pltpu.sync_copy(rinv_hbm, rfull)

    def idx_load(i, q):
      off = (i * NW + wid) * EK
      pltpu.sync_copy(src_hbm.at[pl.ds(off, EK)], srcvs[q])
      pltpu.sync_copy(dst_hbm.at[pl.ds(off, EK)], dstvs[q])
      pltpu.sync_copy(w_hbm.at[pl.ds(off, EK)], wvs[q])

    def issue(q):
      pltpu.async_copy(nx_hbm.at[srcvs[q]], xss[q], sems[q])
      pltpu.async_copy(nx_hbm.at[dstvs[q]], xds[q], sems[q])

    def wait_gather(q):
      pltpu.make_async_copy(nx_hbm.at[srcvs[q]], xss[q], sems[q]).wait()
      pltpu.make_async_copy(nx_hbm.at[dstvs[q]], xds[q], sems[q]).wait()

    def consume(i, q):
      xs, xd = xss[q], xds[q]
      for g in range(EK // L):
        def edge_body(e16, dpack):
          erow = g * L + e16
          acc = jnp.zeros((L,), jnp.float32)
          for j in range(jv):
            sl = pl.ds(j * L, L)
            acc = acc + xs[erow, sl] * xd[erow, sl]
          dot = _lane_sum(acc)
          return jnp.where(lanes == e16, dot, dpack)
        dpack = lax.fori_loop(0, L, edge_body, jnp.zeros((L,), jnp.float32))

        gsl = pl.ds(g * L, L)
        rs = plsc.load_gather(rfull, [srcvs[q][gsl]])
        rd = plsc.load_gather(rfull, [dstvs[q][gsl]])
        cos = dpack * rs * rd
        cd = (1.0 - cos) * 0.5
        outv[gsl] = (wvs[q][gsl] + cd) / (1.0 + cd)

      off = (i * NW + wid) * EK
      pltpu.sync_copy(outv, neww_hbm.at[pl.ds(off, EK)])

    idx_load(0, 0)
    issue(0)
    idx_load(1, 1)
    issue(1)

    def pipe_body(io, _):
      for q in (0, 1):
        i = io * 2 + q
        wait_gather(q)
        consume(i, q)
        idx_load(i + 2, q)
        issue(q)
      return 0
    lax.fori_loop(0, (iters - 2) // 2, pipe_body, 0)

    for q in (0, 1):
      i = iters - 2 + q
      wait_gather(q)
      consume(i, q)

  return k3


def kernel(x, edge_index, w):
  n, d = x.shape
  e = w.shape[0]
  src = edge_index[0].astype(jnp.int32)
  dst = edge_index[1].astype(jnp.int32)

  nblk = ZB * NS
  eblk = EK * NW * 2
  npad = -(-n // nblk) * nblk
  epad = -(-e // eblk) * eblk

  xp = jnp.pad(x, ((0, npad - n), (0, 0)))
  srcp = jnp.pad(src, (0, epad - e))
  dstp = jnp.pad(dst, (0, epad - e))
  wp = jnp.pad(w, (0, epad - e), constant_values=1.0)

  px, pw = _make_k1(npad, epad, d)(xp, srcp, dstp, wp)
  nx_p, rinv_p = _make_k2(npad, d)(xp, px, pw)
  neww_p = _make_k3(npad, epad, d)(nx_p, rinv_p, srcp, dstp, wp)
  return nx_p[:n], neww_p[:e]


# speedup vs baseline: 2.2420x; 2.2420x over previous
import functools

import jax
import jax.numpy as jnp
from jax import lax
from jax.experimental import pallas as pl
from jax.experimental.pallas import tpu as pltpu
from jax.experimental.pallas import tpu_sc as plsc

NC = 2
NS = 16
L = 16
NW = NC * NS
ZB = 80
EK = 128

_GDN = lax.GatherDimensionNumbers(
    offset_dims=(), collapsed_slice_dims=(0,), start_index_map=(0,))


def _perm(v, idx):
  return lax.gather(v, idx[:, None], _GDN, (1,),
                    mode=lax.GatherScatterMode.PROMISE_IN_BOUNDS)


def _lane_sum(v):
  lanes = lax.iota(jnp.int32, L)
  for k in (1, 2, 4, 8):
    v = v + _perm(v, lanes ^ k)
  return v


def _bcast_lane(v, e):
  return _perm(v, jnp.full((L,), e, jnp.int32))


def _rsqrt_vec(v):
  i = lax.bitcast_convert_type(v, jnp.int32)
  i = jnp.int32(0x5F3759DF) - (i >> 1)
  y = lax.bitcast_convert_type(i, jnp.float32)
  for _ in range(3):
    y = y * (1.5 - 0.5 * v * y * y)
  return y


def _mesh():
  return plsc.VectorSubcoreMesh(core_axis_name="c", subcore_axis_name="s",
                                num_cores=NC, num_subcores=NS)


_CPARAMS = pltpu.CompilerParams(needs_layout_passes=False)


def _make_k1(n, e, d):
  iters = e // EK // NW
  zit = n // ZB // NS
  jv = d // L

  @functools.partial(
      pl.kernel,
      out_type=(
          jax.ShapeDtypeStruct((NC * n, d), jnp.float32),
          jax.ShapeDtypeStruct((NW * n,), jnp.float32),
      ),
      mesh=_mesh(),
      compiler_params=_CPARAMS,
      scratch_types=[
          pltpu.VMEM((EK,), jnp.int32),
          pltpu.VMEM((EK,), jnp.int32),
          pltpu.VMEM((EK,), jnp.int32),
          pltpu.VMEM((EK,), jnp.int32),
          pltpu.VMEM((EK,), jnp.float32),
          pltpu.VMEM((EK,), jnp.float32),
          pltpu.VMEM((EK, d), jnp.float32),
          pltpu.VMEM((EK, d), jnp.float32),
          pltpu.VMEM((ZB,), jnp.int32),
          pltpu.VMEM((n,), jnp.float32),
          pltpu.VMEM_SHARED((n, d), jnp.float32),
          pltpu.SemaphoreType.DMA,
          pltpu.SemaphoreType.DMA,
      ],
  )
  def k1(x_hbm, src_hbm, dst_hbm, w_hbm, px_hbm, pw_hbm,
         srcv0, srcv1, dstv0, dstv1, wv0, wv1, rows0, rows1,
         zidx, awacc, accs, sem0, sem1):
    cid = lax.axis_index("c")
    sid = lax.axis_index("s")
    wid = sid * NC + cid
    lanes = lax.iota(jnp.int32, L)
    srcvs = (srcv0, srcv1)
    dstvs = (dstv0, dstv1)
    wvs = (wv0, wv1)
    rowss = (rows0, rows1)
    sems = (sem0, sem1)

    def zdbuf(r, _):
      for j in range(jv):
        rows0[r, pl.ds(j * L, L)] = jnp.zeros((L,), jnp.float32)
      return 0
    lax.fori_loop(0, ZB, zdbuf, 0)

    def zaw(r, _):
      awacc[pl.ds(r * L, L)] = jnp.zeros((L,), jnp.float32)
      return 0
    lax.fori_loop(0, n // L, zaw, 0)

    def set_zidx(base):
      for g in range(ZB // L):
        zidx[pl.ds(g * L, L)] = base + g * L + lanes

    def zchunk(z, _):
      base = (z * NS + sid) * ZB
      set_zidx(base)
      pltpu.sync_copy(rows0.at[pl.ds(0, ZB)], accs.at[zidx])
      return 0
    lax.fori_loop(0, zit, zchunk, 0)
    plsc.subcore_barrier()

    def idx_load(i, q):
      off = (i * NW + wid) * EK
      pltpu.sync_copy(src_hbm.at[pl.ds(off, EK)], srcvs[q])
      pltpu.sync_copy(dst_hbm.at[pl.ds(off, EK)], dstvs[q])
      pltpu.sync_copy(w_hbm.at[pl.ds(off, EK)], wvs[q])

    def issue(q):
      pltpu.async_copy(x_hbm.at[srcvs[q]], rowss[q], sems[q])

    def wait_gather(q):
      pltpu.make_async_copy(x_hbm.at[srcvs[q]], rowss[q], sems[q]).wait()

    def consume(q):
      rows = rowss[q]
      for g in range(EK // L):
        awv = 1.0 - wvs[q][pl.ds(g * L, L)]
        dst16 = dstvs[q][pl.ds(g * L, L)]

        def edge_body(e16, _):
          erow = g * L + e16
          awb = _bcast_lane(awv, e16)
          for j in range(jv):
            sl = pl.ds(j * L, L)
            rows[erow, sl] = rows[erow, sl] * awb
          plsc.addupdate_scatter(awacc, [dst16], awv, mask=lanes == e16)
          return 0
        lax.fori_loop(0, L, edge_body, 0)

      pltpu.sync_copy(rows, accs.at[dstvs[q]], add=True)

    idx_load(0, 0)
    issue(0)
    idx_load(1, 1)
    issue(1)

    def pipe_body(io, _):
      for q in (0, 1):
        i = io * 2 + q
        wait_gather(q)
        consume(q)
        idx_load(i + 2, q)
        issue(q)
      return 0
    lax.fori_loop(0, (iters - 2) // 2, pipe_body, 0)

    for q in (0, 1):
      wait_gather(q)
      consume(q)

    plsc.subcore_barrier()

    def dchunk(z, _):
      base = (z * NS + sid) * ZB
      set_zidx(base)
      pltpu.async_copy(accs.at[zidx], rows0.at[pl.ds(0, ZB)], sem0).wait()
      pltpu.sync_copy(rows0.at[pl.ds(0, ZB)],
                      px_hbm.at[pl.ds(cid * n + base, ZB)])
      return 0
    lax.fori_loop(0, zit, dchunk, 0)

    pltpu.sync_copy(awacc, pw_hbm.at[pl.ds(wid * n, n)])

  return k1


def _make_k2(n, d):
  iters = n // ZB // NW
  jv = d // L

  @functools.partial(
      pl.kernel,
      out_type=(
          jax.ShapeDtypeStruct((n, d), jnp.float32),
          jax.ShapeDtypeStruct((n,), jnp.float32),
      ),
      mesh=_mesh(),
      compiler_params=_CPARAMS,
      scratch_types=[
          pltpu.VMEM((ZB, d), jnp.float32),
          pltpu.VMEM((ZB, d), jnp.float32),
          pltpu.VMEM((ZB, d), jnp.float32),
          pltpu.VMEM((NW * ZB,), jnp.float32),
          pltpu.VMEM((ZB, d), jnp.float32),
          pltpu.VMEM((ZB,), jnp.float32),
      ],
  )
  def k2(x_hbm, px_hbm, pw_hbm, nx_hbm, rinv_hbm,
         xv, p0v, p1v, pwv, outv, rv):
    cid = lax.axis_index("c")
    sid = lax.axis_index("s")
    wid = sid * NC + cid
    lanes = lax.iota(jnp.int32, L)

    def chunk_body(i, _):
      ro = (i * NW + wid) * ZB
      sl_rows = pl.ds(ro, ZB)
      pltpu.sync_copy(x_hbm.at[sl_rows], xv)
      pltpu.sync_copy(px_hbm.at[pl.ds(ro, ZB)], p0v)
      pltpu.sync_copy(px_hbm.at[pl.ds(n + ro, ZB)], p1v)
      for t in range(NW):
        pltpu.sync_copy(pw_hbm.at[pl.ds(t * n + ro, ZB)],
                        pwv.at[pl.ds(t * ZB, ZB)])

      for g in range(ZB // L):
        saw = jnp.zeros((L,), jnp.float32)
        for t in range(NW):
          saw = saw + pwv[pl.ds(t * ZB + g * L, L)]

        def node_body(e16, rpack):
          r = g * L + e16
          den = 1.0 + _bcast_lane(saw, e16)
          ss = jnp.zeros((L,), jnp.float32)
          for j in range(jv):
            sl = pl.ds(j * L, L)
            num = (xv[r, sl] + p0v[r, sl] + p1v[r, sl]) / den
            outv[r, sl] = num
            ss = ss + num * num
          rr = jnp.minimum(_rsqrt_vec(_lane_sum(ss)), 1e8)
          return jnp.where(lanes == e16, rr, rpack)
        rpack = lax.fori_loop(0, L, node_body, jnp.zeros((L,), jnp.float32))
        rv[pl.ds(g * L, L)] = rpack

      pltpu.sync_copy(outv, nx_hbm.at[sl_rows])
      pltpu.sync_copy(rv, rinv_hbm.at[sl_rows])
      return 0
    lax.fori_loop(0, iters, chunk_body, 0)

  return k2


def _make_k3(n, e, d):
  iters = e // EK // NW
  jv = d // L

  @functools.partial(
      pl.kernel,
      out_type=jax.ShapeDtypeStruct((e,), jnp.float32),
      mesh=_mesh(),
      compiler_params=_CPARAMS,
      scratch_types=[
          pltpu.VMEM((EK,), jnp.int32),
          pltpu.VMEM((EK,), jnp.int32),
          pltpu.VMEM((EK,), jnp.int32),
          pltpu.VMEM((EK,), jnp.int32),
          pltpu.VMEM((EK,), jnp.float32),
          pltpu.VMEM((EK,), jnp.float32),
          pltpu.VMEM((EK, d), jnp.float32),
          pltpu.VMEM((EK, d), jnp.float32),
          pltpu.VMEM((EK, d), jnp.float32),
          pltpu.VMEM((EK, d), jnp.float32),
          pltpu.VMEM((EK,), jnp.float32),
          pltpu.VMEM((n,), jnp.float32),
          pltpu.SemaphoreType.DMA,
          pltpu.SemaphoreType.DMA,
      ],
  )
  def k3(nx_hbm, rinv_hbm, src_hbm, dst_hbm, w_hbm, neww_hbm,
         srcv0, srcv1, dstv0, dstv1, wv0, wv1, xs0, xs1, xd0, xd1,
         outv, rfull, sem0, sem1):
    cid = lax.axis_index("c")
    sid = lax.axis_index("s")
    wid = sid * NC + cid
    lanes = lax.iota(jnp.int32, L)
    srcvs = (srcv0, srcv1)
    dstvs = (dstv0, dstv1)
    wvs = (wv0, wv1)
    xss = (xs0, xs1)
    xds = (xd0, xd1)
    sems = (sem0, sem1)

    pltpu.sync_copy(rinv_hbm, rfull)

    def idx_load(i, q):
      off = (i * NW + wid) * EK
      pltpu.sync_copy(src_hbm.at[pl.ds(off, EK)], srcvs[q])
      pltpu.sync_copy(dst_hbm.at[pl.ds(off, EK)], dstvs[q])
      pltpu.sync_copy(w_hbm.at[pl.ds(off, EK)], wvs[q])

    def issue(q):
      pltpu.async_copy(nx_hbm.at[srcvs[q]], xss[q], sems[q])
      pltpu.async_copy(nx_hbm.at[dstvs[q]], xds[q], sems[q])

    def wait_gather(q):
      pltpu.make_async_copy(nx_hbm.at[srcvs[q]], xss[q], sems[q]).wait()
      pltpu.make_async_copy(nx_hbm.at[dstvs[q]], xds[q], sems[q]).wait()

    def consume(i, q):
      xs, xd = xss[q], xds[q]
      for g in range(EK // L):
        def edge_body(e16, dpack):
          erow = g * L + e16
          acc = jnp.zeros((L,), jnp.float32)
          for j in range(jv):
            sl = pl.ds(j * L, L)
            acc = acc + xs[erow, sl] * xd[erow, sl]
          dot = _lane_sum(acc)
          return jnp.where(lanes == e16, dot, dpack)
        dpack = lax.fori_loop(0, L, edge_body, jnp.zeros((L,), jnp.float32))

        gsl = pl.ds(g * L, L)
        rs = plsc.load_gather(rfull, [srcvs[q][gsl]])
        rd = plsc.load_gather(rfull, [dstvs[q][gsl]])
        cos = dpack * rs * rd
        cd = (1.0 - cos) * 0.5
        outv[gsl] = (wvs[q][gsl] + cd) / (1.0 + cd)

      off = (i * NW + wid) * EK
      pltpu.sync_copy(outv, neww_hbm.at[pl.ds(off, EK)])

    idx_load(0, 0)
    issue(0)
    idx_load(1, 1)
    issue(1)

    def pipe_body(io, _):
      for q in (0, 1):
        i = io * 2 + q
        wait_gather(q)
        consume(i, q)
        idx_load(i + 2, q)
        issue(q)
      return 0
    lax.fori_loop(0, (iters - 2) // 2, pipe_body, 0)

    for q in (0, 1):
      i = iters - 2 + q
      wait_gather(q)
      consume(i, q)

  return k3


def kernel(x, edge_index, w):
  n, d = x.shape
  e = w.shape[0]
  src = edge_index[0].astype(jnp.int32)
  dst = edge_index[1].astype(jnp.int32)

  nblk = ZB * NS
  eblk = EK * NW * 2
  npad = -(-n // nblk) * nblk
  epad = -(-e // eblk) * eblk

  xp = jnp.pad(x, ((0, npad - n), (0, 0)))
  spread = (jnp.arange(epad - e, dtype=jnp.int32) * 37) % n
  srcp = jnp.concatenate([src, spread])
  dstp = jnp.concatenate([dst, spread])
  wp = jnp.pad(w, (0, epad - e), constant_values=1.0)

  px, pw = _make_k1(npad, epad, d)(xp, srcp, dstp, wp)
  nx_p, rinv_p = _make_k2(npad, d)(xp, px, pw)
  neww_p = _make_k3(npad, epad, d)(nx_p, rinv_p, srcp, dstp, wp)
  return nx_p[:n], neww_p[:e]


# speedup vs baseline: 2.4301x; 1.0839x over previous
import functools

import jax
import jax.numpy as jnp
from jax import lax
from jax.experimental import pallas as pl
from jax.experimental.pallas import tpu as pltpu
from jax.experimental.pallas import tpu_sc as plsc

NC = 2
NS = 16
L = 16
NW = NC * NS
ZB = 80
EK = 128

_GDN = lax.GatherDimensionNumbers(
    offset_dims=(), collapsed_slice_dims=(0,), start_index_map=(0,))


def _perm(v, idx):
  return lax.gather(v, idx[:, None], _GDN, (1,),
                    mode=lax.GatherScatterMode.PROMISE_IN_BOUNDS)


def _lane_sum(v):
  lanes = lax.iota(jnp.int32, L)
  for k in (1, 2, 4, 8):
    v = v + _perm(v, lanes ^ k)
  return v


def _bcast_lane(v, e):
  return _perm(v, jnp.full((L,), e, jnp.int32))


def _rsqrt_vec(v):
  i = lax.bitcast_convert_type(v, jnp.int32)
  i = jnp.int32(0x5F3759DF) - (i >> 1)
  y = lax.bitcast_convert_type(i, jnp.float32)
  for _ in range(3):
    y = y * (1.5 - 0.5 * v * y * y)
  return y


def _mesh():
  return plsc.VectorSubcoreMesh(core_axis_name="c", subcore_axis_name="s",
                                num_cores=NC, num_subcores=NS)


_CPARAMS = pltpu.CompilerParams(needs_layout_passes=False)


def _make_k1(n, e, d):
  iters = e // EK // NW
  zit = n // ZB // NS
  jv = d // L
  nr = n // d
  sh = d.bit_length() - 1
  assert d == (1 << sh) and nr == ZB and nr % NS == 0

  @functools.partial(
      pl.kernel,
      out_type=(
          jax.ShapeDtypeStruct((NC * n, d), jnp.float32),
          jax.ShapeDtypeStruct((NC * nr, d), jnp.float32),
      ),
      mesh=_mesh(),
      compiler_params=_CPARAMS,
      scratch_types=[
          pltpu.VMEM((EK,), jnp.int32),
          pltpu.VMEM((EK,), jnp.int32),
          pltpu.VMEM((EK,), jnp.int32),
          pltpu.VMEM((EK,), jnp.int32),
          pltpu.VMEM((EK,), jnp.float32),
          pltpu.VMEM((EK,), jnp.float32),
          pltpu.VMEM((EK, d), jnp.float32),
          pltpu.VMEM((EK, d), jnp.float32),
          pltpu.VMEM((ZB,), jnp.int32),
          pltpu.VMEM((L,), jnp.int32),
          pltpu.VMEM((L,), jnp.int32),
          pltpu.VMEM((nr, d), jnp.float32),
          pltpu.VMEM_SHARED((n, d), jnp.float32),
          pltpu.VMEM_SHARED((nr, d), jnp.float32),
          pltpu.SemaphoreType.DMA,
          pltpu.SemaphoreType.DMA,
      ],
  )
  def k1(x_hbm, src_hbm, dst_hbm, w_hbm, px_hbm, pw_hbm,
         srcv0, srcv1, dstv0, dstv1, wv0, wv1, rows0, rows1,
         zidx, aidx, aidx2, awacc, accs, accw2, sem0, sem1):
    cid = lax.axis_index("c")
    sid = lax.axis_index("s")
    wid = sid * NC + cid
    lanes = lax.iota(jnp.int32, L)
    srcvs = (srcv0, srcv1)
    dstvs = (dstv0, dstv1)
    wvs = (wv0, wv1)
    rowss = (rows0, rows1)
    sems = (sem0, sem1)

    def zdbuf(r, _):
      for j in range(jv):
        rows0[r, pl.ds(j * L, L)] = jnp.zeros((L,), jnp.float32)
      return 0
    lax.fori_loop(0, ZB, zdbuf, 0)

    def zaw(r, _):
      for j in range(jv):
        awacc[r, pl.ds(j * L, L)] = jnp.zeros((L,), jnp.float32)
      return 0
    lax.fori_loop(0, nr, zaw, 0)

    share = jnp.minimum(sid * (nr // NS) + lanes, nr - 1)
    aidx[:] = share
    aidx2[:] = cid * nr + share
    pltpu.sync_copy(rows0.at[pl.ds(0, L)], accw2.at[aidx])

    def set_zidx(base):
      for g in range(ZB // L):
        zidx[pl.ds(g * L, L)] = base + g * L + lanes

    def zchunk(z, _):
      base = (z * NS + sid) * ZB
      set_zidx(base)
      pltpu.sync_copy(rows0.at[pl.ds(0, ZB)], accs.at[zidx])
      return 0
    lax.fori_loop(0, zit, zchunk, 0)
    plsc.subcore_barrier()

    def idx_load(i, q):
      off = (i * NW + wid) * EK
      pltpu.sync_copy(src_hbm.at[pl.ds(off, EK)], srcvs[q])
      pltpu.sync_copy(dst_hbm.at[pl.ds(off, EK)], dstvs[q])
      pltpu.sync_copy(w_hbm.at[pl.ds(off, EK)], wvs[q])

    def issue(q):
      pltpu.async_copy(x_hbm.at[srcvs[q]], rowss[q], sems[q])

    def wait_gather(q):
      pltpu.make_async_copy(x_hbm.at[srcvs[q]], rowss[q], sems[q]).wait()

    def consume(q):
      rows = rowss[q]
      for g in range(EK // L):
        awv = 1.0 - wvs[q][pl.ds(g * L, L)]
        dst16 = dstvs[q][pl.ds(g * L, L)]

        def edge_body(e16, _):
          erow = g * L + e16
          awb = _bcast_lane(awv, e16)
          for j in range(jv):
            sl = pl.ds(j * L, L)
            rows[erow, sl] = rows[erow, sl] * awb
          plsc.addupdate_scatter(awacc, [dst16 >> sh, dst16 & (d - 1)],
                                 awv, mask=lanes == e16)
          return 0
        lax.fori_loop(0, L, edge_body, 0)

      pltpu.sync_copy(rows, accs.at[dstvs[q]], add=True)

    idx_load(0, 0)
    issue(0)
    idx_load(1, 1)
    issue(1)

    def pipe_body(io, _):
      for q in (0, 1):
        i = io * 2 + q
        wait_gather(q)
        consume(q)
        idx_load(i + 2, q)
        issue(q)
      return 0
    lax.fori_loop(0, (iters - 2) // 2, pipe_body, 0)

    for q in (0, 1):
      wait_gather(q)
      consume(q)

    set_zidx(0)
    pltpu.sync_copy(awacc, accw2.at[zidx], add=True)

    plsc.subcore_barrier()

    pltpu.async_copy(accw2.at[aidx], rows0.at[pl.ds(0, L)], sem0).wait()
    pltpu.sync_copy(rows0.at[pl.ds(0, L)], pw_hbm.at[aidx2])

    def dchunk(z, _):
      base = (z * NS + sid) * ZB
      set_zidx(base)
      pltpu.async_copy(accs.at[zidx], rows0.at[pl.ds(0, ZB)], sem0).wait()
      pltpu.sync_copy(rows0.at[pl.ds(0, ZB)],
                      px_hbm.at[pl.ds(cid * n + base, ZB)])
      return 0
    lax.fori_loop(0, zit, dchunk, 0)

  return k1


def _make_k2(n, d):
  iters = n // ZB // NW
  jv = d // L

  @functools.partial(
      pl.kernel,
      out_type=(
          jax.ShapeDtypeStruct((n, d), jnp.float32),
          jax.ShapeDtypeStruct((n,), jnp.float32),
      ),
      mesh=_mesh(),
      compiler_params=_CPARAMS,
      scratch_types=[
          pltpu.VMEM((ZB, d), jnp.float32),
          pltpu.VMEM((ZB, d), jnp.float32),
          pltpu.VMEM((ZB, d), jnp.float32),
          pltpu.VMEM((NC * ZB,), jnp.float32),
          pltpu.VMEM((ZB, d), jnp.float32),
          pltpu.VMEM((ZB,), jnp.float32),
      ],
  )
  def k2(x_hbm, px_hbm, pw_hbm, nx_hbm, rinv_hbm,
         xv, p0v, p1v, pwv, outv, rv):
    cid = lax.axis_index("c")
    sid = lax.axis_index("s")
    wid = sid * NC + cid
    lanes = lax.iota(jnp.int32, L)

    def chunk_body(i, _):
      ro = (i * NW + wid) * ZB
      sl_rows = pl.ds(ro, ZB)
      pltpu.sync_copy(x_hbm.at[sl_rows], xv)
      pltpu.sync_copy(px_hbm.at[pl.ds(ro, ZB)], p0v)
      pltpu.sync_copy(px_hbm.at[pl.ds(n + ro, ZB)], p1v)
      for t in range(NC):
        pltpu.sync_copy(pw_hbm.at[pl.ds(t * n + ro, ZB)],
                        pwv.at[pl.ds(t * ZB, ZB)])

      for g in range(ZB // L):
        saw = jnp.zeros((L,), jnp.float32)
        for t in range(NC):
          saw = saw + pwv[pl.ds(t * ZB + g * L, L)]

        def node_body(e16, rpack):
          r = g * L + e16
          den = 1.0 + _bcast_lane(saw, e16)
          ss = jnp.zeros((L,), jnp.float32)
          for j in range(jv):
            sl = pl.ds(j * L, L)
            num = (xv[r, sl] + p0v[r, sl] + p1v[r, sl]) / den
            outv[r, sl] = num
            ss = ss + num * num
          rr = jnp.minimum(_rsqrt_vec(_lane_sum(ss)), 1e8)
          return jnp.where(lanes == e16, rr, rpack)
        rpack = lax.fori_loop(0, L, node_body, jnp.zeros((L,), jnp.float32))
        rv[pl.ds(g * L, L)] = rpack

      pltpu.sync_copy(outv, nx_hbm.at[sl_rows])
      pltpu.sync_copy(rv, rinv_hbm.at[sl_rows])
      return 0
    lax.fori_loop(0, iters, chunk_body, 0)

  return k2


def _make_k3(n, e, d):
  iters = e // EK // NW
  jv = d // L

  @functools.partial(
      pl.kernel,
      out_type=jax.ShapeDtypeStruct((e,), jnp.float32),
      mesh=_mesh(),
      compiler_params=_CPARAMS,
      scratch_types=[
          pltpu.VMEM((EK,), jnp.int32),
          pltpu.VMEM((EK,), jnp.int32),
          pltpu.VMEM((EK,), jnp.int32),
          pltpu.VMEM((EK,), jnp.int32),
          pltpu.VMEM((EK,), jnp.float32),
          pltpu.VMEM((EK,), jnp.float32),
          pltpu.VMEM((EK, d), jnp.float32),
          pltpu.VMEM((EK, d), jnp.float32),
          pltpu.VMEM((EK, d), jnp.float32),
          pltpu.VMEM((EK, d), jnp.float32),
          pltpu.VMEM((EK,), jnp.float32),
          pltpu.VMEM((n,), jnp.float32),
          pltpu.SemaphoreType.DMA,
          pltpu.SemaphoreType.DMA,
      ],
  )
  def k3(nx_hbm, rinv_hbm, src_hbm, dst_hbm, w_hbm, neww_hbm,
         srcv0, srcv1, dstv0, dstv1, wv0, wv1, xs0, xs1, xd0, xd1,
         outv, rfull, sem0, sem1):
    cid = lax.axis_index("c")
    sid = lax.axis_index("s")
    wid = sid * NC + cid
    lanes = lax.iota(jnp.int32, L)
    srcvs = (srcv0, srcv1)
    dstvs = (dstv0, dstv1)
    wvs = (wv0, wv1)
    xss = (xs0, xs1)
    xds = (xd0, xd1)
    sems = (sem0, sem1)

    pltpu.sync_copy(rinv_hbm, rfull)

    def idx_load(i, q):
      off = (i * NW + wid) * EK
      pltpu.sync_copy(src_hbm.at[pl.ds(off, EK)], srcvs[q])
      pltpu.sync_copy(dst_hbm.at[pl.ds(off, EK)], dstvs[q])
      pltpu.sync_copy(w_hbm.at[pl.ds(off, EK)], wvs[q])

    def issue(q):
      pltpu.async_copy(nx_hbm.at[srcvs[q]], xss[q], sems[q])
      pltpu.async_copy(nx_hbm.at[dstvs[q]], xds[q], sems[q])

    def wait_gather(q):
      pltpu.make_async_copy(nx_hbm.at[srcvs[q]], xss[q], sems[q]).wait()
      pltpu.make_async_copy(nx_hbm.at[dstvs[q]], xds[q], sems[q]).wait()

    def consume(i, q):
      xs, xd = xss[q], xds[q]
      for g in range(EK // L):
        def edge_body(e16, dpack):
          erow = g * L + e16
          acc = jnp.zeros((L,), jnp.float32)
          for j in range(jv):
            sl = pl.ds(j * L, L)
            acc = acc + xs[erow, sl] * xd[erow, sl]
          dot = _lane_sum(acc)
          return jnp.where(lanes == e16, dot, dpack)
        dpack = lax.fori_loop(0, L, edge_body, jnp.zeros((L,), jnp.float32))

        gsl = pl.ds(g * L, L)
        rs = plsc.load_gather(rfull, [srcvs[q][gsl]])
        rd = plsc.load_gather(rfull, [dstvs[q][gsl]])
        cos = dpack * rs * rd
        cd = (1.0 - cos) * 0.5
        outv[gsl] = (wvs[q][gsl] + cd) / (1.0 + cd)

      off = (i * NW + wid) * EK
      pltpu.sync_copy(outv, neww_hbm.at[pl.ds(off, EK)])

    idx_load(0, 0)
    issue(0)
    idx_load(1, 1)
    issue(1)

    def pipe_body(io, _):
      for q in (0, 1):
        i = io * 2 + q
        wait_gather(q)
        consume(i, q)
        idx_load(i + 2, q)
        issue(q)
      return 0
    lax.fori_loop(0, (iters - 2) // 2, pipe_body, 0)

    for q in (0, 1):
      i = iters - 2 + q
      wait_gather(q)
      consume(i, q)

  return k3


def kernel(x, edge_index, w):
  n, d = x.shape
  e = w.shape[0]
  src = edge_index[0].astype(jnp.int32)
  dst = edge_index[1].astype(jnp.int32)

  nblk = ZB * NS
  eblk = EK * NW * 2
  npad = -(-n // nblk) * nblk
  epad = -(-e // eblk) * eblk

  xp = jnp.pad(x, ((0, npad - n), (0, 0)))
  spread = (jnp.arange(epad - e, dtype=jnp.int32) * 37) % n
  srcp = jnp.concatenate([src, spread])
  dstp = jnp.concatenate([dst, spread])
  wp = jnp.pad(w, (0, epad - e), constant_values=1.0)

  px, pw = _make_k1(npad, epad, d)(xp, srcp, dstp, wp)
  nx_p, rinv_p = _make_k2(npad, d)(xp, px, pw.reshape(-1))
  neww_p = _make_k3(npad, epad, d)(nx_p, rinv_p, srcp, dstp, wp)
  return nx_p[:n], neww_p[:e]


# speedup vs baseline: 2.8397x; 1.1686x over previous
import functools

import jax
import jax.numpy as jnp
from jax import lax
from jax.experimental import pallas as pl
from jax.experimental.pallas import tpu as pltpu
from jax.experimental.pallas import tpu_sc as plsc

NC = 2
NS = 16
L = 16
NW = NC * NS
ZB = 80
EK = 128

_GDN = lax.GatherDimensionNumbers(
    offset_dims=(), collapsed_slice_dims=(0,), start_index_map=(0,))


def _perm(v, idx):
  return lax.gather(v, idx[:, None], _GDN, (1,),
                    mode=lax.GatherScatterMode.PROMISE_IN_BOUNDS)


def _lane_sum(v):
  lanes = lax.iota(jnp.int32, L)
  for k in (1, 2, 4, 8):
    v = v + _perm(v, lanes ^ k)
  return v


def _bcast_lane(v, e):
  return _perm(v, jnp.full((L,), e, jnp.int32))


def _rsqrt_vec(v):
  i = lax.bitcast_convert_type(v, jnp.int32)
  i = jnp.int32(0x5F3759DF) - (i >> 1)
  y = lax.bitcast_convert_type(i, jnp.float32)
  for _ in range(3):
    y = y * (1.5 - 0.5 * v * y * y)
  return y


def _mesh():
  return plsc.VectorSubcoreMesh(core_axis_name="c", subcore_axis_name="s",
                                num_cores=NC, num_subcores=NS)


_CPARAMS = pltpu.CompilerParams(needs_layout_passes=False)


def _make_k1(n, e, d):
  iters = e // EK // NW
  zit = n // ZB // NS
  jv = d // L
  nr = n // d
  sh = d.bit_length() - 1
  assert d == (1 << sh) and nr == ZB and nr % NS == 0

  @functools.partial(
      pl.kernel,
      out_type=(
          jax.ShapeDtypeStruct((NC * n, d), jnp.float32),
          jax.ShapeDtypeStruct((NC * nr, d), jnp.float32),
      ),
      mesh=_mesh(),
      compiler_params=_CPARAMS,
      scratch_types=[
          pltpu.VMEM((EK,), jnp.int32),
          pltpu.VMEM((EK,), jnp.int32),
          pltpu.VMEM((EK,), jnp.int32),
          pltpu.VMEM((EK,), jnp.int32),
          pltpu.VMEM((EK,), jnp.float32),
          pltpu.VMEM((EK,), jnp.float32),
          pltpu.VMEM((EK, d), jnp.float32),
          pltpu.VMEM((EK, d), jnp.float32),
          pltpu.VMEM((ZB,), jnp.int32),
          pltpu.VMEM((L,), jnp.int32),
          pltpu.VMEM((L,), jnp.int32),
          pltpu.VMEM((nr, d), jnp.float32),
          pltpu.VMEM_SHARED((n, d), jnp.float32),
          pltpu.VMEM_SHARED((nr, d), jnp.float32),
          pltpu.SemaphoreType.DMA,
          pltpu.SemaphoreType.DMA,
      ],
  )
  def k1(x_hbm, src_hbm, dst_hbm, w_hbm, px_hbm, pw_hbm,
         srcv0, srcv1, dstv0, dstv1, wv0, wv1, rows0, rows1,
         zidx, aidx, aidx2, awacc, accs, accw2, sem0, sem1):
    cid = lax.axis_index("c")
    sid = lax.axis_index("s")
    wid = sid * NC + cid
    lanes = lax.iota(jnp.int32, L)
    srcvs = (srcv0, srcv1)
    dstvs = (dstv0, dstv1)
    wvs = (wv0, wv1)
    rowss = (rows0, rows1)
    sems = (sem0, sem1)

    def zdbuf(r, _):
      for j in range(jv):
        rows0[r, pl.ds(j * L, L)] = jnp.zeros((L,), jnp.float32)
      return 0
    lax.fori_loop(0, ZB, zdbuf, 0)

    def zaw(r, _):
      for j in range(jv):
        awacc[r, pl.ds(j * L, L)] = jnp.zeros((L,), jnp.float32)
      return 0
    lax.fori_loop(0, nr, zaw, 0)

    share = jnp.minimum(sid * (nr // NS) + lanes, nr - 1)
    aidx[:] = share
    aidx2[:] = cid * nr + share
    pltpu.sync_copy(rows0.at[pl.ds(0, L)], accw2.at[aidx])

    def set_zidx(base):
      for g in range(ZB // L):
        zidx[pl.ds(g * L, L)] = base + g * L + lanes

    def zchunk(z, _):
      base = (z * NS + sid) * ZB
      set_zidx(base)
      pltpu.sync_copy(rows0.at[pl.ds(0, ZB)], accs.at[zidx])
      return 0
    lax.fori_loop(0, zit, zchunk, 0)
    plsc.subcore_barrier()

    def idx_load(i, q):
      off = (i * NW + wid) * EK
      pltpu.sync_copy(src_hbm.at[pl.ds(off, EK)], srcvs[q])
      pltpu.sync_copy(dst_hbm.at[pl.ds(off, EK)], dstvs[q])
      pltpu.sync_copy(w_hbm.at[pl.ds(off, EK)], wvs[q])

    def issue(q):
      pltpu.async_copy(x_hbm.at[srcvs[q]], rowss[q], sems[q])

    def wait_gather(q):
      pltpu.make_async_copy(x_hbm.at[srcvs[q]], rowss[q], sems[q]).wait()

    def consume(q):
      rows = rowss[q]
      for g in range(EK // L):
        awv = 1.0 - wvs[q][pl.ds(g * L, L)]
        dst16 = dstvs[q][pl.ds(g * L, L)]

        def edge_body(e16, _):
          erow = g * L + e16
          awb = _bcast_lane(awv, e16)
          for j in range(jv):
            sl = pl.ds(j * L, L)
            rows[erow, sl] = rows[erow, sl] * awb
          plsc.addupdate_scatter(awacc, [dst16 >> sh, dst16 & (d - 1)],
                                 awv, mask=lanes == e16)
          return 0
        lax.fori_loop(0, L, edge_body, 0)

      pltpu.sync_copy(rows, accs.at[dstvs[q]], add=True)

    idx_load(0, 0)
    issue(0)
    idx_load(1, 1)
    issue(1)

    def pipe_body(io, _):
      for q in (0, 1):
        i = io * 2 + q
        wait_gather(q)
        consume(q)
        idx_load(i + 2, q)
        issue(q)
      return 0
    lax.fori_loop(0, (iters - 2) // 2, pipe_body, 0)

    for q in (0, 1):
      wait_gather(q)
      consume(q)

    set_zidx(0)
    pltpu.sync_copy(awacc, accw2.at[zidx], add=True)

    plsc.subcore_barrier()

    pltpu.async_copy(accw2.at[aidx], rows0.at[pl.ds(0, L)], sem0).wait()
    pltpu.sync_copy(rows0.at[pl.ds(0, L)], pw_hbm.at[aidx2])

    def dchunk(z, _):
      base = (z * NS + sid) * ZB
      set_zidx(base)
      pltpu.async_copy(accs.at[zidx], rows0.at[pl.ds(0, ZB)], sem0).wait()
      pltpu.sync_copy(rows0.at[pl.ds(0, ZB)],
                      px_hbm.at[pl.ds(cid * n + base, ZB)])
      return 0
    lax.fori_loop(0, zit, dchunk, 0)

  return k1


def _make_k2(n, d):
  iters = n // ZB // NW
  jv = d // L

  @functools.partial(
      pl.kernel,
      out_type=(
          jax.ShapeDtypeStruct((n, d), jnp.float32),
          jax.ShapeDtypeStruct((n,), jnp.float32),
      ),
      mesh=_mesh(),
      compiler_params=_CPARAMS,
      scratch_types=[
          pltpu.VMEM((ZB, d), jnp.float32),
          pltpu.VMEM((ZB, d), jnp.float32),
          pltpu.VMEM((ZB, d), jnp.float32),
          pltpu.VMEM((NC * ZB,), jnp.float32),
          pltpu.VMEM((ZB, d), jnp.float32),
          pltpu.VMEM((ZB,), jnp.float32),
      ],
  )
  def k2(x_hbm, px_hbm, pw_hbm, nx_hbm, rinv_hbm,
         xv, p0v, p1v, pwv, outv, rv):
    cid = lax.axis_index("c")
    sid = lax.axis_index("s")
    wid = sid * NC + cid
    lanes = lax.iota(jnp.int32, L)

    def chunk_body(i, _):
      ro = (i * NW + wid) * ZB
      sl_rows = pl.ds(ro, ZB)
      pltpu.sync_copy(x_hbm.at[sl_rows], xv)
      pltpu.sync_copy(px_hbm.at[pl.ds(ro, ZB)], p0v)
      pltpu.sync_copy(px_hbm.at[pl.ds(n + ro, ZB)], p1v)
      for t in range(NC):
        pltpu.sync_copy(pw_hbm.at[pl.ds(t * n + ro, ZB)],
                        pwv.at[pl.ds(t * ZB, ZB)])

      for g in range(ZB // L):
        saw = jnp.zeros((L,), jnp.float32)
        for t in range(NC):
          saw = saw + pwv[pl.ds(t * ZB + g * L, L)]

        def node_body(e16, rpack):
          r = g * L + e16
          den = 1.0 + _bcast_lane(saw, e16)
          ss = jnp.zeros((L,), jnp.float32)
          for j in range(jv):
            sl = pl.ds(j * L, L)
            num = (xv[r, sl] + p0v[r, sl] + p1v[r, sl]) / den
            outv[r, sl] = num
            ss = ss + num * num
          rr = jnp.minimum(_rsqrt_vec(_lane_sum(ss)), 1e8)
          return jnp.where(lanes == e16, rr, rpack)
        rpack = lax.fori_loop(0, L, node_body, jnp.zeros((L,), jnp.float32))
        rv[pl.ds(g * L, L)] = rpack

      pltpu.sync_copy(outv, nx_hbm.at[sl_rows])
      pltpu.sync_copy(rv, rinv_hbm.at[sl_rows])
      return 0
    lax.fori_loop(0, iters, chunk_body, 0)

  return k2


def _make_k3(n, e, d):
  iters = e // EK // NW
  jv = d // L
  te = iters * EK

  @functools.partial(
      pl.kernel,
      out_type=jax.ShapeDtypeStruct((e,), jnp.float32),
      mesh=_mesh(),
      compiler_params=_CPARAMS,
      scratch_types=[
          pltpu.VMEM((te,), jnp.int32),
          pltpu.VMEM((te,), jnp.int32),
          pltpu.VMEM((te,), jnp.float32),
          pltpu.VMEM((EK, d), jnp.float32),
          pltpu.VMEM((EK, d), jnp.float32),
          pltpu.VMEM((EK, d), jnp.float32),
          pltpu.VMEM((EK, d), jnp.float32),
          pltpu.VMEM((EK,), jnp.float32),
          pltpu.VMEM((n,), jnp.float32),
          pltpu.SemaphoreType.DMA,
          pltpu.SemaphoreType.DMA,
      ],
  )
  def k3(nx_hbm, rinv_hbm, src_hbm, dst_hbm, w_hbm, neww_hbm,
         srcb, dstb, wb, xs0, xs1, xd0, xd1, outv, rfull, sem0, sem1):
    cid = lax.axis_index("c")
    sid = lax.axis_index("s")
    wid = sid * NC + cid
    lanes = lax.iota(jnp.int32, L)
    xss = (xs0, xs1)
    xds = (xd0, xd1)
    sems = (sem0, sem1)
    base = wid * te

    pltpu.sync_copy(rinv_hbm, rfull)
    pltpu.sync_copy(src_hbm.at[pl.ds(base, te)], srcb)
    pltpu.sync_copy(dst_hbm.at[pl.ds(base, te)], dstb)
    pltpu.sync_copy(w_hbm.at[pl.ds(base, te)], wb)

    def issue(i, q):
      lo = i * EK
      pltpu.async_copy(nx_hbm.at[srcb.at[pl.ds(lo, EK)]], xss[q], sems[q])
      pltpu.async_copy(nx_hbm.at[dstb.at[pl.ds(lo, EK)]], xds[q], sems[q])

    def wait_gather(i, q):
      lo = i * EK
      pltpu.make_async_copy(
          nx_hbm.at[srcb.at[pl.ds(lo, EK)]], xss[q], sems[q]).wait()
      pltpu.make_async_copy(
          nx_hbm.at[dstb.at[pl.ds(lo, EK)]], xds[q], sems[q]).wait()

    def consume(i, q):
      xs, xd = xss[q], xds[q]
      lo = i * EK
      for g in range(EK // L):
        def edge_body(e16, dpack):
          erow = g * L + e16
          acc = jnp.zeros((L,), jnp.float32)
          for j in range(jv):
            sl = pl.ds(j * L, L)
            acc = acc + xs[erow, sl] * xd[erow, sl]
          dot = _lane_sum(acc)
          return jnp.where(lanes == e16, dot, dpack)
        dpack = lax.fori_loop(0, L, edge_body, jnp.zeros((L,), jnp.float32))

        gsl = pl.ds(g * L, L)
        bsl = pl.ds(lo + g * L, L)
        rs = plsc.load_gather(rfull, [srcb[bsl]])
        rd = plsc.load_gather(rfull, [dstb[bsl]])
        cos = dpack * rs * rd
        cd = (1.0 - cos) * 0.5
        outv[gsl] = (wb[bsl] + cd) / (1.0 + cd)

      pltpu.sync_copy(outv, neww_hbm.at[pl.ds(base + lo, EK)])

    issue(0, 0)
    issue(1, 1)

    def pipe_body(io, _):
      for q in (0, 1):
        i = io * 2 + q
        wait_gather(i, q)
        consume(i, q)
        issue(i + 2, q)
      return 0
    lax.fori_loop(0, (iters - 2) // 2, pipe_body, 0)

    for q in (0, 1):
      i = iters - 2 + q
      wait_gather(i, q)
      consume(i, q)

  return k3


def kernel(x, edge_index, w):
  n, d = x.shape
  e = w.shape[0]
  src = edge_index[0].astype(jnp.int32)
  dst = edge_index[1].astype(jnp.int32)

  nblk = ZB * NS
  eblk = EK * NW * 2
  npad = -(-n // nblk) * nblk
  epad = -(-e // eblk) * eblk

  xp = jnp.pad(x, ((0, npad - n), (0, 0)))
  spread = (jnp.arange(epad - e, dtype=jnp.int32) * 37) % n
  srcp = jnp.concatenate([src, spread])
  dstp = jnp.concatenate([dst, spread])
  wp = jnp.pad(w, (0, epad - e), constant_values=1.0)

  px, pw = _make_k1(npad, epad, d)(xp, srcp, dstp, wp)
  nx_p, rinv_p = _make_k2(npad, d)(xp, px, pw.reshape(-1))
  neww_p = _make_k3(npad, epad, d)(nx_p, rinv_p, srcp, dstp, wp)
  return nx_p[:n], neww_p[:e]
